# Initial kernel scaffold; baseline (speedup 1.0000x reference)
import jax
import jax.numpy as jnp
from jax.experimental import pallas as pl


def kernel(nodes_feature, params, edge_index, options_cls):
    raise NotImplementedError("write your pallas kernel here")


# speedup vs baseline: 7.3068x; 7.3068x over previous
import functools

import jax
import jax.numpy as jnp
from jax import lax
from jax.experimental import pallas as pl
from jax.experimental.pallas import tpu as pltpu
from jax.experimental.pallas import tpu_sc as plsc

N = 10000
E = 20000
B = 32
O = 4
H = 768

NCH = 6
CW = 128
NTILES = 16
RPT = 632
ACC_ROWS = NTILES * RPT
PIECES = [(0, 128), (128, 128), (256, 128), (384, 128)]
TAIL_FULL = (512, 120)
TAIL_LAST = (512, 8)
BSZ = 128
NBT = 10
EPAD = NTILES * NBT * BSZ
KF = 2
BLKN = 1000


def _ln(x, g, b, eps=1e-5):
    mu = jnp.mean(x, -1, keepdims=True)
    var = jnp.mean((x - mu) ** 2, -1, keepdims=True)
    return (x - mu) / jnp.sqrt(var + eps) * g + b


def _enc_math(x, wq, bq, wk, bk, wv, bv, wo, bo, g1, c1, w1, b1, w2, b2,
              g2, c2, nh, L):
    R, d = x.shape
    dh = d // nh
    q = jnp.dot(x, wq, preferred_element_type=jnp.float32) + bq
    k = jnp.dot(x, wk, preferred_element_type=jnp.float32) + bk
    v = jnp.dot(x, wv, preferred_element_type=jnp.float32) + bv
    ri = lax.broadcasted_iota(jnp.int32, (R, R), 0) // L
    ci = lax.broadcasted_iota(jnp.int32, (R, R), 1) // L
    mask = ri == ci
    outs = []
    for h in range(nh):
        qh = q[:, h * dh:(h + 1) * dh]
        kh = k[:, h * dh:(h + 1) * dh]
        vh = v[:, h * dh:(h + 1) * dh]
        s = lax.dot_general(qh, kh, (((1,), (1,)), ((), ())),
                            preferred_element_type=jnp.float32)
        s = s / jnp.sqrt(float(dh))
        s = jnp.where(mask, s, -1e30)
        s = s - jnp.max(s, -1, keepdims=True)
        e = jnp.exp(s)
        a = e / jnp.sum(e, -1, keepdims=True)
        outs.append(jnp.dot(a, vh, preferred_element_type=jnp.float32))
    o = jnp.concatenate(outs, axis=1)
    attn = jnp.dot(o, wo, preferred_element_type=jnp.float32) + bo
    h1 = _ln(x + attn, g1, c1)
    f = jnp.maximum(jnp.dot(h1, w1, preferred_element_type=jnp.float32) + b1, 0.0)
    f = jnp.dot(f, w2, preferred_element_type=jnp.float32) + b2
    return _ln(h1 + f, g2, c2)


def _enc_args(x, p):
    d = x.shape[1]
    r1 = lambda a: a.reshape(1, -1)
    return [x, p['Wq'], r1(p['bq']), p['Wk'], r1(p['bk']), p['Wv'],
            r1(p['bv']), p['Wo'], r1(p['bo']), r1(p['ln1_g']), r1(p['ln1_b']),
            p['W1'], r1(p['b1']), p['W2'], r1(p['b2']), r1(p['ln2_g']),
            r1(p['ln2_b'])]


def _enc1_body(x_ref, wq, bq, wk, bk, wv, bv, wo, bo, g1, c1, w1, b1, w2, b2,
               g2, c2, o_ref):
    o_ref[...] = _enc_math(x_ref[...], wq[...], bq[...], wk[...], bk[...],
                           wv[...], bv[...], wo[...], bo[...], g1[...],
                           c1[...], w1[...], b1[...], w2[...], b2[...],
                           g2[...], c2[...], nh=2, L=O)


def _enc1_tc(x, p):
    return pl.pallas_call(
        _enc1_body,
        out_shape=jax.ShapeDtypeStruct(x.shape, jnp.float32),
    )(*_enc_args(x, p))


def _readout_body(x_ref, wq, bq, wk, bk, wv, bv, wo, bo, g1, c1, w1, b1, w2,
                  b2, g2, c2, l1w, l1b, l2w, l2b, o_ref):
    enc = _enc_math(x_ref[...], wq[...], bq[...], wk[...], bk[...], wv[...],
                    bv[...], wo[...], bo[...], g1[...], c1[...], w1[...],
                    b1[...], w2[...], b2[...], g2[...], c2[...], nh=2, L=O)
    t = jnp.tanh(jnp.dot(enc, l1w[...], preferred_element_type=jnp.float32)
                 + l1b[...])
    o_ref[...] = (jnp.dot(t, l2w[...], preferred_element_type=jnp.float32)
                  + l2b[...])


def _readout_tc(x, p_enc, lin1, lin2):
    args = _enc_args(x, p_enc) + [lin1['W'], lin1['b'].reshape(1, -1),
                                  lin2['W'], lin2['b'].reshape(1, -1)]
    return pl.pallas_call(
        _readout_body,
        out_shape=jax.ShapeDtypeStruct((x.shape[0], 1), jnp.float32),
    )(*args)


def _rgcn_body(nodes_ref, agg_ref, deg_ref, root_ref, w0_ref, b_ref, o_ref):
    nodes = nodes_ref[...]
    agg = jnp.concatenate([agg_ref[c] for c in range(NCH)], axis=1)
    scale = 1.0 / jnp.maximum(deg_ref[...][:, 0:1], 1.0)
    acc = jnp.dot(nodes, root_ref[...], preferred_element_type=jnp.float32)
    acc = acc + jnp.dot(agg * scale, w0_ref[...],
                        preferred_element_type=jnp.float32)
    acc = acc + b_ref[...]
    for c in range(NCH):
        o_ref[c] = acc[:, c * CW:(c + 1) * CW]


def _rgcn_tc(nodes, agg1_c, deg, rg):
    return pl.pallas_call(
        _rgcn_body,
        grid=(N // BLKN,),
        in_specs=[
            pl.BlockSpec((BLKN, H), lambda i: (i, 0)),
            pl.BlockSpec((NCH, BLKN, CW), lambda i: (0, i, 0)),
            pl.BlockSpec((BLKN, CW), lambda i: (i, 0)),
            pl.BlockSpec((H, H), lambda i: (0, 0)),
            pl.BlockSpec((H, H), lambda i: (0, 0)),
            pl.BlockSpec((1, H), lambda i: (0, 0)),
        ],
        out_specs=pl.BlockSpec((NCH, BLKN, CW), lambda i: (0, i, 0)),
        out_shape=jax.ShapeDtypeStruct((NCH, N, CW), jnp.float32),
    )(nodes, agg1_c, deg, rg['root'], rg['W'][0], rg['b'].reshape(1, H))


def _gconv_body(out_ref, agg2_ref, wroot_ref, wrel_ref, b_ref, o_ref):
    o = jnp.concatenate([out_ref[c] for c in range(NCH)], axis=1)
    a2 = jnp.concatenate([agg2_ref[c] for c in range(NCH)], axis=1)
    h2 = jnp.dot(a2, wrel_ref[...], preferred_element_type=jnp.float32)
    h2 = h2 + jnp.dot(o, wroot_ref[...], preferred_element_type=jnp.float32)
    o_ref[...] = h2 + b_ref[...]


def _gconv_tc(out_c, agg2_c, gc):
    return pl.pallas_call(
        _gconv_body,
        grid=(N // BLKN,),
        in_specs=[
            pl.BlockSpec((NCH, BLKN, CW), lambda i: (0, i, 0)),
            pl.BlockSpec((NCH, BLKN, CW), lambda i: (0, i, 0)),
            pl.BlockSpec((H, 64), lambda i: (0, 0)),
            pl.BlockSpec((H, 64), lambda i: (0, 0)),
            pl.BlockSpec((1, 64), lambda i: (0, 0)),
        ],
        out_specs=pl.BlockSpec((BLKN, 64), lambda i: (i, 0)),
        out_shape=jax.ShapeDtypeStruct((N, 64), jnp.float32),
    )(out_c, agg2_c, gc['W_root'], gc['W_rel'], gc['b'].reshape(1, 64))


def _sc_segsum(x_c, src_p, dst_p, with_deg):
    mesh = plsc.VectorSubcoreMesh(core_axis_name="c", subcore_axis_name="s")
    out_type = [jax.ShapeDtypeStruct((NCH, N, CW), jnp.float32)]
    if with_deg:
        out_type.append(jax.ShapeDtypeStruct((N, CW), jnp.float32))
    scratch = [
        pltpu.VMEM((KF, BSZ), jnp.int32),
        pltpu.VMEM((KF, BSZ), jnp.int32),
        pltpu.VMEM((KF, BSZ, CW), jnp.float32),
        pltpu.SemaphoreType.DMA,
        pltpu.SemaphoreType.DMA,
        pltpu.VMEM_SHARED((ACC_ROWS, CW), jnp.float32),
    ]

    def body(*refs):
        if with_deg:
            (x_ref, src_ref, dst_ref, out_ref, deg_ref, idx_s, idx_d, rows,
             gsem, ssem, acc) = refs
        else:
            (x_ref, src_ref, dst_ref, out_ref, idx_s, idx_d, rows,
             gsem, ssem, acc) = refs
        c = lax.axis_index("c")
        s = lax.axis_index("s")
        base_row = s * RPT
        ebase = s * (NBT * BSZ)

        def fill_rows0(val):
            vec = jnp.full((16,), val, jnp.float32)

            def frow(i, carry):
                rows[0, i // 8, pl.ds((i % 8) * 16, 16)] = vec
                return carry
            lax.fori_loop(0, BSZ * (CW // 16), frow, 0)

        def zero_acc():
            fill_rows0(0.0)
            for lo, ln in PIECES + [TAIL_FULL]:
                pltpu.sync_copy(rows.at[0, pl.ds(0, ln)],
                                acc.at[pl.ds(base_row + lo, ln)])
            plsc.subcore_barrier()

        def writeback(dst_hbm):
            plsc.subcore_barrier()

            def wb_piece(lo, ln):
                pltpu.sync_copy(acc.at[pl.ds(base_row + lo, ln)],
                                rows.at[0, pl.ds(0, ln)])
                pltpu.sync_copy(rows.at[0, pl.ds(0, ln)],
                                dst_hbm.at[pl.ds(base_row + lo, ln)])
            for lo, ln in PIECES:
                wb_piece(lo, ln)

            @pl.when(s < NTILES - 1)
            def _():
                wb_piece(*TAIL_FULL)

            @pl.when(s == NTILES - 1)
            def _():
                wb_piece(*TAIL_LAST)

        def process_chunk(ck):
            x_ck = x_ref.at[ck]
            zero_acc()

            def round_body(r, carry):
                goff = ebase + r * (KF * BSZ)
                gds = []
                for b2 in range(KF):
                    off = goff + b2 * BSZ
                    pltpu.sync_copy(src_ref.at[pl.ds(off, BSZ)], idx_s.at[b2])
                    pltpu.sync_copy(dst_ref.at[pl.ds(off, BSZ)], idx_d.at[b2])
                    gds.append(pltpu.async_copy(
                        x_ck.at[idx_s.at[b2]], rows.at[b2], gsem))
                for g in gds:
                    g.wait()
                sds = []
                for b2 in range(KF):
                    sds.append(pltpu.async_copy(
                        rows.at[b2], acc.at[idx_d.at[b2]], ssem, add=True))
                for sd in sds:
                    sd.wait()
                return carry
            lax.fori_loop(0, NBT // KF, round_body, 0)
            writeback(out_ref.at[ck])

        def process_deg():
            zero_acc()
            fill_rows0(1.0)

            def round_body(r, carry):
                off = ebase + r * BSZ
                pltpu.sync_copy(dst_ref.at[pl.ds(off, BSZ)], idx_d.at[0])
                pltpu.async_copy(rows.at[0], acc.at[idx_d.at[0]],
                                 ssem, add=True).wait()
                return carry
            lax.fori_loop(0, NBT, round_body, 0)
            writeback(deg_ref)

        for cv in range(2):
            @pl.when(c == cv)
            def _(cv=cv):
                if with_deg and cv == 0:
                    process_deg()
                for k in range(NCH // 2):
                    process_chunk(cv * (NCH // 2) + k)

    fn = pl.kernel(body, out_type=tuple(out_type), mesh=mesh,
                   scratch_types=tuple(scratch))
    return fn(x_c, src_p, dst_p)


def kernel(nodes_feature, params, edge_index, options_cls):
    p = params
    opt_raw = nodes_feature[options_cls]
    opt_mut = _enc1_tc(opt_raw, p['enc1'])
    nodes = nodes_feature.at[options_cls].set(opt_mut)
    nodes_c = nodes.reshape(N, NCH, CW).transpose(1, 0, 2)

    src = edge_index[0]
    dst = edge_index[1]
    pad = EPAD - E
    src_p = jnp.concatenate([src, jnp.zeros((pad,), jnp.int32)])
    dst_p = jnp.concatenate([dst, jnp.full((pad,), N, jnp.int32)])

    agg1_c, deg = _sc_segsum(nodes_c, src_p, dst_p, with_deg=True)
    out_c = _rgcn_tc(nodes, agg1_c, deg, p['rgcn'])
    agg2_c, = _sc_segsum(out_c, src_p, dst_p, with_deg=False)
    h2 = _gconv_tc(out_c, agg2_c, p['gconv'])

    res = _readout_tc(h2[options_cls], p['enc2'], p['lin1'], p['lin2'])
    return res.reshape(B, O, 1)


# speedup vs baseline: 7.6314x; 1.0444x over previous
import functools

import jax
import jax.numpy as jnp
from jax import lax
from jax.experimental import pallas as pl
from jax.experimental.pallas import tpu as pltpu
from jax.experimental.pallas import tpu_sc as plsc

N = 10000
E = 20000
B = 32
O = 4
H = 768

NCH = 6
CW = 128
NTILES = 16
RPT = 632
ACC_ROWS = NTILES * RPT
PIECES = [(0, 128), (128, 128), (256, 128), (384, 128)]
TAIL_FULL = (512, 120)
TAIL_LAST = (512, 8)
BSZ = 128
NBT = 10
EPAD = NTILES * NBT * BSZ
KF = 2
BLKN = 1000


def _ln(x, g, b, eps=1e-5):
    mu = jnp.mean(x, -1, keepdims=True)
    var = jnp.mean((x - mu) ** 2, -1, keepdims=True)
    return (x - mu) / jnp.sqrt(var + eps) * g + b


def _enc_math(x, wq, bq, wk, bk, wv, bv, wo, bo, g1, c1, w1, b1, w2, b2,
              g2, c2, nh, L):
    R, d = x.shape
    dh = d // nh
    q = jnp.dot(x, wq, preferred_element_type=jnp.float32) + bq
    k = jnp.dot(x, wk, preferred_element_type=jnp.float32) + bk
    v = jnp.dot(x, wv, preferred_element_type=jnp.float32) + bv
    ri = lax.broadcasted_iota(jnp.int32, (R, R), 0) // L
    ci = lax.broadcasted_iota(jnp.int32, (R, R), 1) // L
    mask = ri == ci
    outs = []
    for h in range(nh):
        qh = q[:, h * dh:(h + 1) * dh]
        kh = k[:, h * dh:(h + 1) * dh]
        vh = v[:, h * dh:(h + 1) * dh]
        s = lax.dot_general(qh, kh, (((1,), (1,)), ((), ())),
                            preferred_element_type=jnp.float32)
        s = s / jnp.sqrt(float(dh))
        s = jnp.where(mask, s, -1e30)
        s = s - jnp.max(s, -1, keepdims=True)
        e = jnp.exp(s)
        a = e / jnp.sum(e, -1, keepdims=True)
        outs.append(jnp.dot(a, vh, preferred_element_type=jnp.float32))
    o = jnp.concatenate(outs, axis=1)
    attn = jnp.dot(o, wo, preferred_element_type=jnp.float32) + bo
    h1 = _ln(x + attn, g1, c1)
    f = jnp.maximum(jnp.dot(h1, w1, preferred_element_type=jnp.float32) + b1, 0.0)
    f = jnp.dot(f, w2, preferred_element_type=jnp.float32) + b2
    return _ln(h1 + f, g2, c2)


def _enc_args(x, p):
    d = x.shape[1]
    r1 = lambda a: a.reshape(1, -1)
    return [x, p['Wq'], r1(p['bq']), p['Wk'], r1(p['bk']), p['Wv'],
            r1(p['bv']), p['Wo'], r1(p['bo']), r1(p['ln1_g']), r1(p['ln1_b']),
            p['W1'], r1(p['b1']), p['W2'], r1(p['b2']), r1(p['ln2_g']),
            r1(p['ln2_b'])]


def _enc1_body(x_ref, wq, bq, wk, bk, wv, bv, wo, bo, g1, c1, w1, b1, w2, b2,
               g2, c2, o_ref):
    o_ref[...] = _enc_math(x_ref[...], wq[...], bq[...], wk[...], bk[...],
                           wv[...], bv[...], wo[...], bo[...], g1[...],
                           c1[...], w1[...], b1[...], w2[...], b2[...],
                           g2[...], c2[...], nh=2, L=O)


def _enc1_tc(x, p):
    return pl.pallas_call(
        _enc1_body,
        out_shape=jax.ShapeDtypeStruct(x.shape, jnp.float32),
    )(*_enc_args(x, p))


def _readout_body(x_ref, wq, bq, wk, bk, wv, bv, wo, bo, g1, c1, w1, b1, w2,
                  b2, g2, c2, l1w, l1b, l2w, l2b, o_ref):
    enc = _enc_math(x_ref[...], wq[...], bq[...], wk[...], bk[...], wv[...],
                    bv[...], wo[...], bo[...], g1[...], c1[...], w1[...],
                    b1[...], w2[...], b2[...], g2[...], c2[...], nh=2, L=O)
    t = jnp.tanh(jnp.dot(enc, l1w[...], preferred_element_type=jnp.float32)
                 + l1b[...])
    o_ref[...] = (jnp.dot(t, l2w[...], preferred_element_type=jnp.float32)
                  + l2b[...])


def _readout_tc(x, p_enc, lin1, lin2):
    args = _enc_args(x, p_enc) + [lin1['W'], lin1['b'].reshape(1, -1),
                                  lin2['W'], lin2['b'].reshape(1, -1)]
    return pl.pallas_call(
        _readout_body,
        out_shape=jax.ShapeDtypeStruct((x.shape[0], 1), jnp.float32),
    )(*args)


def _rgcn_body(nodes_ref, agg_ref, deg_ref, root_ref, w0_ref, b_ref, o_ref):
    nodes = nodes_ref[...]
    agg = jnp.concatenate([agg_ref[c] for c in range(NCH)], axis=1)
    scale = 1.0 / jnp.maximum(deg_ref[...][:, 0:1], 1.0)
    acc = jnp.dot(nodes, root_ref[...], preferred_element_type=jnp.float32)
    acc = acc + jnp.dot(agg * scale, w0_ref[...],
                        preferred_element_type=jnp.float32)
    acc = acc + b_ref[...]
    for c in range(NCH):
        o_ref[c] = acc[:, c * CW:(c + 1) * CW]


def _rgcn_tc(nodes, agg1_c, deg, rg):
    return pl.pallas_call(
        _rgcn_body,
        grid=(N // BLKN,),
        in_specs=[
            pl.BlockSpec((BLKN, H), lambda i: (i, 0)),
            pl.BlockSpec((NCH, BLKN, CW), lambda i: (0, i, 0)),
            pl.BlockSpec((BLKN, CW), lambda i: (i, 0)),
            pl.BlockSpec((H, H), lambda i: (0, 0)),
            pl.BlockSpec((H, H), lambda i: (0, 0)),
            pl.BlockSpec((1, H), lambda i: (0, 0)),
        ],
        out_specs=pl.BlockSpec((NCH, BLKN, CW), lambda i: (0, i, 0)),
        out_shape=jax.ShapeDtypeStruct((NCH, N, CW), jnp.float32),
    )(nodes, agg1_c, deg, rg['root'], rg['W'][0], rg['b'].reshape(1, H))


def _gconv_body(out_ref, agg2_ref, wroot_ref, wrel_ref, b_ref, o_ref):
    o = jnp.concatenate([out_ref[c] for c in range(NCH)], axis=1)
    a2 = jnp.concatenate([agg2_ref[c] for c in range(NCH)], axis=1)
    h2 = jnp.dot(a2, wrel_ref[...], preferred_element_type=jnp.float32)
    h2 = h2 + jnp.dot(o, wroot_ref[...], preferred_element_type=jnp.float32)
    o_ref[...] = h2 + b_ref[...]


def _gconv_tc(out_c, agg2_c, gc):
    return pl.pallas_call(
        _gconv_body,
        grid=(N // BLKN,),
        in_specs=[
            pl.BlockSpec((NCH, BLKN, CW), lambda i: (0, i, 0)),
            pl.BlockSpec((NCH, BLKN, CW), lambda i: (0, i, 0)),
            pl.BlockSpec((H, 64), lambda i: (0, 0)),
            pl.BlockSpec((H, 64), lambda i: (0, 0)),
            pl.BlockSpec((1, 64), lambda i: (0, 0)),
        ],
        out_specs=pl.BlockSpec((BLKN, 64), lambda i: (i, 0)),
        out_shape=jax.ShapeDtypeStruct((N, 64), jnp.float32),
    )(out_c, agg2_c, gc['W_root'], gc['W_rel'], gc['b'].reshape(1, 64))


def _sc_segsum(x_c, src_p, dst_p, with_deg):
    mesh = plsc.VectorSubcoreMesh(core_axis_name="c", subcore_axis_name="s")
    out_type = [jax.ShapeDtypeStruct((NCH, N, CW), jnp.float32)]
    if with_deg:
        out_type.append(jax.ShapeDtypeStruct((N, CW), jnp.float32))
    scratch = [
        pltpu.VMEM((NBT, BSZ), jnp.int32),
        pltpu.VMEM((NBT, BSZ), jnp.int32),
        pltpu.VMEM((KF, BSZ, CW), jnp.float32),
        pltpu.SemaphoreType.DMA,
        pltpu.SemaphoreType.DMA,
        pltpu.SemaphoreType.DMA,
        pltpu.SemaphoreType.DMA,
        pltpu.VMEM_SHARED((ACC_ROWS, CW), jnp.float32),
    ]

    def body(*refs):
        if with_deg:
            (x_ref, src_ref, dst_ref, out_ref, deg_ref, idx_s, idx_d, rows,
             gsem0, gsem1, ssem0, ssem1, acc) = refs
        else:
            (x_ref, src_ref, dst_ref, out_ref, idx_s, idx_d, rows,
             gsem0, gsem1, ssem0, ssem1, acc) = refs
        gsems = (gsem0, gsem1)
        ssems = (ssem0, ssem1)
        c = lax.axis_index("c")
        s = lax.axis_index("s")
        base_row = s * RPT
        ebase = s * (NBT * BSZ)

        for j in range(NBT):
            pltpu.sync_copy(src_ref.at[pl.ds(ebase + j * BSZ, BSZ)],
                            idx_s.at[j])
            pltpu.sync_copy(dst_ref.at[pl.ds(ebase + j * BSZ, BSZ)],
                            idx_d.at[j])

        def fill_rows0(val):
            vec = jnp.full((16,), val, jnp.float32)

            def frow(i, carry):
                rows[0, i // 8, pl.ds((i % 8) * 16, 16)] = vec
                return carry
            lax.fori_loop(0, BSZ * (CW // 16), frow, 0)

        def zero_acc():
            fill_rows0(0.0)
            for lo, ln in PIECES + [TAIL_FULL]:
                pltpu.sync_copy(rows.at[0, pl.ds(0, ln)],
                                acc.at[pl.ds(base_row + lo, ln)])
            plsc.subcore_barrier()

        def writeback(dst_hbm):
            plsc.subcore_barrier()

            def wb_piece(lo, ln):
                pltpu.sync_copy(acc.at[pl.ds(base_row + lo, ln)],
                                rows.at[0, pl.ds(0, ln)])
                pltpu.sync_copy(rows.at[0, pl.ds(0, ln)],
                                dst_hbm.at[pl.ds(base_row + lo, ln)])
            for lo, ln in PIECES:
                wb_piece(lo, ln)

            @pl.when(s < NTILES - 1)
            def _():
                wb_piece(*TAIL_FULL)

            @pl.when(s == NTILES - 1)
            def _():
                wb_piece(*TAIL_LAST)

        def process_chunk(ck):
            x_ck = x_ref.at[ck]
            zero_acc()

            def g_start(j, slot):
                return pltpu.async_copy(x_ck.at[idx_s.at[j]],
                                        rows.at[slot], gsems[slot])

            def g_wait(slot):
                pltpu.make_async_copy(x_ck.at[idx_s.at[0]], rows.at[slot],
                                      gsems[slot]).wait()

            def s_start(j, slot):
                return pltpu.async_copy(rows.at[slot], acc.at[idx_d.at[j]],
                                        ssems[slot], add=True)

            def s_wait(slot):
                pltpu.make_async_copy(rows.at[slot], acc.at[idx_d.at[0]],
                                      ssems[slot]).wait()

            g_start(0, 0)
            nit = NBT // 2

            def iter_body(i, carry):
                @pl.when(i > 0)
                def _():
                    s_wait(1)
                g_start(2 * i + 1, 1)
                g_wait(0)
                s_start(2 * i, 0)
                g_wait(1)
                s_start(2 * i + 1, 1)
                s_wait(0)

                @pl.when(i < nit - 1)
                def _():
                    g_start(2 * i + 2, 0)
                return carry
            lax.fori_loop(0, nit, iter_body, 0)
            s_wait(1)
            writeback(out_ref.at[ck])

        def process_deg():
            zero_acc()
            fill_rows0(1.0)

            def round_body(r, carry):
                pltpu.async_copy(rows.at[0], acc.at[idx_d.at[r]],
                                 ssems[0], add=True).wait()
                return carry
            lax.fori_loop(0, NBT, round_body, 0)
            writeback(deg_ref)

        for cv in range(2):
            @pl.when(c == cv)
            def _(cv=cv):
                if with_deg and cv == 0:
                    process_deg()
                for k in range(NCH // 2):
                    process_chunk(cv * (NCH // 2) + k)

    fn = pl.kernel(body, out_type=tuple(out_type), mesh=mesh,
                   scratch_types=tuple(scratch))
    return fn(x_c, src_p, dst_p)


def kernel(nodes_feature, params, edge_index, options_cls):
    p = params
    opt_raw = nodes_feature[options_cls]
    opt_mut = _enc1_tc(opt_raw, p['enc1'])
    nodes = nodes_feature.at[options_cls].set(opt_mut)
    nodes_c = nodes.reshape(N, NCH, CW).transpose(1, 0, 2)

    src = edge_index[0]
    dst = edge_index[1]
    pad = EPAD - E
    src_p = jnp.concatenate([src, jnp.zeros((pad,), jnp.int32)])
    dst_p = jnp.concatenate([dst, jnp.full((pad,), N, jnp.int32)])

    agg1_c, deg = _sc_segsum(nodes_c, src_p, dst_p, with_deg=True)
    out_c = _rgcn_tc(nodes, agg1_c, deg, p['rgcn'])
    agg2_c, = _sc_segsum(out_c, src_p, dst_p, with_deg=False)
    h2 = _gconv_tc(out_c, agg2_c, p['gconv'])

    res = _readout_tc(h2[options_cls], p['enc2'], p['lin1'], p['lin2'])
    return res.reshape(B, O, 1)
